# baseline (speedup 1.0000x reference)
import functools

import jax
import jax.numpy as jnp
from jax import lax
from jax.experimental import pallas as pl
from jax.experimental.pallas import tpu as pltpu
from jax.experimental.pallas import tpu_sc as plsc

N = 10000
D = 128
E = 320000
NC = 2
NS = 16
NW = NC * NS
EPT = E // NW
K = 50
CHUNKS = EPT // K
NB = 3
NG = CHUNKS // NB
DR = 1000
NDT = N // DR
CW = 8


def _sc_body(x_hbm, row_hbm, col_hbm, zrow_hbm, zcnt_hbm,
             acc_out, cnt_out,
             row_idx, col_idx, bufs, ones_v,
             acc_sh, cnt_sh, gsems, ssems):
    cid = lax.axis_index("c")
    sid = lax.axis_index("s")
    wid = sid * NC + cid

    pltpu.sync_copy(row_hbm.at[pl.ds(wid * CHUNKS, CHUNKS)], row_idx)
    pltpu.sync_copy(col_hbm.at[pl.ds(wid * CHUNKS, CHUNKS)], col_idx)

    def fill_ones(i, _):
        ones_v[i] = jnp.ones((CW,), jnp.float32)
        return 0
    lax.fori_loop(0, K, fill_ones, 0)

    rbase = sid * DR

    @pl.when(sid < NDT)
    def _init():
        pltpu.sync_copy(zrow_hbm, acc_sh.at[pl.ds(rbase, DR)])
        pltpu.sync_copy(zcnt_hbm, cnt_sh.at[pl.ds(rbase, DR)])

    plsc.subcore_barrier()

    def g_start(j, b):
        pltpu.async_copy(x_hbm.at[row_idx.at[j]], bufs[b], gsems[b])

    def g_wait(b):
        pltpu.make_async_copy(x_hbm.at[row_idx.at[0]], bufs[b],
                              gsems[b]).wait()

    def s_start(j, b):
        pltpu.async_copy(bufs[b], acc_sh.at[col_idx.at[j]], ssems[b],
                         add=True)
        pltpu.async_copy(ones_v, cnt_sh.at[col_idx.at[j]], ssems[b],
                         add=True)

    def s_wait(b):
        pltpu.make_async_copy(bufs[b], acc_sh.at[col_idx.at[0]],
                              ssems[b]).wait()
        pltpu.make_async_copy(ones_v, cnt_sh.at[col_idx.at[0]],
                              ssems[b]).wait()

    for b in range(NB):
        g_start(b, b)

    def body(t, _):
        j0 = NB * t
        for b in range(NB):
            g_wait(b)
            s_start(j0 + b, b)
        for b in range(NB):
            s_wait(b)

            @pl.when(j0 + NB + b < CHUNKS)
            def _():
                g_start(j0 + NB + b, b)
        return 0
    lax.fori_loop(0, NG, body, 0)

    for b in range(CHUNKS - NB * NG):
        g_wait(b)
        s_start(NB * NG + b, b)
    for b in range(CHUNKS - NB * NG):
        s_wait(b)

    plsc.subcore_barrier()

    @pl.when(sid < NDT)
    def _drain():
        pltpu.sync_copy(acc_sh.at[pl.ds(rbase, DR)],
                        acc_out.at[cid, pl.ds(rbase, DR)])
        pltpu.sync_copy(cnt_sh.at[pl.ds(rbase, DR)],
                        cnt_out.at[cid, pl.ds(rbase, DR)])


_sc_kernel = functools.partial(
    pl.kernel,
    out_type=(
        jax.ShapeDtypeStruct((NC, N, D), jnp.float32),
        jax.ShapeDtypeStruct((NC, N, CW), jnp.float32),
    ),
    mesh=plsc.VectorSubcoreMesh(core_axis_name="c", subcore_axis_name="s"),
    scratch_types=[
        pltpu.VMEM((CHUNKS, K), jnp.int32),
        pltpu.VMEM((CHUNKS, K), jnp.int32),
        [pltpu.VMEM((K, D), jnp.float32) for _ in range(NB)],
        pltpu.VMEM((K, CW), jnp.float32),
        pltpu.VMEM_SHARED((N, D), jnp.float32),
        pltpu.VMEM_SHARED((N, CW), jnp.float32),
        [pltpu.SemaphoreType.DMA for _ in range(NB)],
        [pltpu.SemaphoreType.DMA for _ in range(NB)],
    ],
    compiler_params=pltpu.CompilerParams(use_tc_tiling_on_sc=False),
)(_sc_body)


BN = 1000


def _tc_body(x_ref, acc_ref, cnt_ref, out_ref):
    cnt = jnp.sum(cnt_ref[...], axis=(0, 2)) * (1.0 / CW)
    s = acc_ref[0] + acc_ref[1]
    inv = 1.0 / jnp.maximum(cnt, 1.0)
    out_ref[:, :D] = s * inv[:, None]
    mask = jnp.where(cnt > 0.0, 1.0, 0.0)
    out_ref[:, D:] = x_ref[...] * mask[:, None]


_tc_finalize = pl.pallas_call(
    _tc_body,
    grid=(N // BN,),
    in_specs=[
        pl.BlockSpec((BN, D), lambda i: (i, 0)),
        pl.BlockSpec((NC, BN, D), lambda i: (0, i, 0)),
        pl.BlockSpec((NC, BN, CW), lambda i: (0, i, 0)),
    ],
    out_specs=pl.BlockSpec((BN, 2 * D), lambda i: (i, 0)),
    out_shape=jax.ShapeDtypeStruct((N, 2 * D), jnp.float32),
)


@jax.jit
def kernel(x, es):
    col = es[0].astype(jnp.int32).reshape(E // K, K)
    row = es[1].astype(jnp.int32).reshape(E // K, K)
    zrow = jnp.zeros((DR, D), jnp.float32)
    zcnt = jnp.zeros((DR, CW), jnp.float32)
    acc, cnt = _sc_kernel(x, row, col, zrow, zcnt)
    return _tc_finalize(x, acc, cnt)
